# baseline (speedup 1.0000x reference)
import functools

import jax
import jax.numpy as jnp
from jax import lax
from jax.experimental import pallas as pl
from jax.experimental.pallas import tpu as pltpu
from jax.experimental.pallas import tpu_sc as plsc

_BATCH = 16384
_D = 32
_LANES = 16
_CHUNK = 128


def _make_sc_logits():
    mesh = plsc.VectorSubcoreMesh(core_axis_name="c", subcore_axis_name="s")
    nw = mesh.num_cores * mesh.num_subcores
    bpw = _BATCH // nw
    nch = bpw // _CHUNK
    groups = bpw // _LANES

    @functools.partial(
        pl.kernel,
        out_type=jax.ShapeDtypeStruct((_BATCH,), jnp.float32),
        mesh=mesh,
        scratch_types=[
            pltpu.VMEM((nch, _CHUNK), jnp.int32),
            pltpu.VMEM((nch, _CHUNK), jnp.int32),
            pltpu.VMEM((bpw, _D), jnp.float32),
            pltpu.VMEM((bpw, _D), jnp.float32),
            pltpu.VMEM((_D,), jnp.float32),
            pltpu.VMEM((bpw,), jnp.float32),
            pltpu.SemaphoreType.DMA,
        ],
        compiler_params=pltpu.CompilerParams(
            needs_layout_passes=False, use_tc_tiling_on_sc=False),
    )
    def sc_logits(user_hbm, item_hbm, ut_hbm, it_hbm, w_hbm, out_hbm,
                  idx_u, idx_v, u_rows, v_rows, w_v, lg, sem):
        wid = lax.axis_index("s") * mesh.num_cores + lax.axis_index("c")

        pltpu.sync_copy(user_hbm.at[pl.ds(wid * nch, nch)], idx_u)
        pltpu.sync_copy(item_hbm.at[pl.ds(wid * nch, nch)], idx_v)
        pltpu.sync_copy(w_hbm, w_v)

        copies = []
        for j in range(nch):
            dst = pl.ds(j * _CHUNK, _CHUNK)
            copies.append(pltpu.async_copy(ut_hbm.at[idx_u.at[j]], u_rows.at[dst], sem))
            copies.append(pltpu.async_copy(it_hbm.at[idx_v.at[j]], v_rows.at[dst], sem))
        for c in copies:
            c.wait()

        w_lo = w_v[pl.ds(0, _LANES)]
        w_hi = w_v[pl.ds(_LANES, _LANES)]
        wds = [w_lo[d] for d in range(_LANES)] + [w_hi[d] for d in range(_LANES)]
        lane = lax.iota(jnp.int32, _LANES)

        def body(g, carry):
            rows = g * _LANES + lane
            acc = jnp.zeros((_LANES,), jnp.float32)
            for d in range(_D):
                cols = jnp.full((_LANES,), d, jnp.int32)
                uu = plsc.load_gather(u_rows, [rows, cols])
                vv = plsc.load_gather(v_rows, [rows, cols])
                acc = acc + (uu * vv) * wds[d]
            lg[pl.ds(g * _LANES, _LANES)] = acc
            return carry

        lax.fori_loop(0, groups, body, 0)
        pltpu.sync_copy(lg, out_hbm.at[pl.ds(wid * bpw, bpw)])

    return sc_logits


_sc_logits = _make_sc_logits()


def _bce_body(b_ref, x_ref, t_ref, o_ref):
    x = x_ref[...] + b_ref[0]
    t = t_ref[...]
    z = jnp.maximum(x, 0.0) - x * t + jnp.log1p(jnp.exp(-jnp.abs(x)))
    o_ref[0, 0] = jnp.sum(z) * (1.0 / _BATCH)


_bce = pl.pallas_call(
    _bce_body,
    out_shape=jax.ShapeDtypeStruct((1, 1), jnp.float32),
    in_specs=[
        pl.BlockSpec(memory_space=pltpu.SMEM),
        pl.BlockSpec(memory_space=pltpu.VMEM),
        pl.BlockSpec(memory_space=pltpu.VMEM),
    ],
    out_specs=pl.BlockSpec(memory_space=pltpu.SMEM),
)


def kernel(user, item, label, user_table, item_table, W, b):
    user2d = user.reshape(-1, _CHUNK)
    item2d = item.reshape(-1, _CHUNK)
    logits = _sc_logits(user2d, item2d, user_table, item_table, W.reshape(-1))
    loss = _bce(b, logits.reshape(_CHUNK, _CHUNK), label.reshape(_CHUNK, _CHUNK))
    return loss[0, 0]
